# Initial kernel scaffold; baseline (speedup 1.0000x reference)
import jax
import jax.numpy as jnp
from jax.experimental import pallas as pl


def kernel(x, row_embed):
    raise NotImplementedError("write your pallas kernel here")


# speedup vs baseline: 1.3830x; 1.3830x over previous
import functools

import jax
import jax.numpy as jnp
from jax import lax
from jax.experimental import pallas as pl
from jax.experimental.pallas import tpu as pltpu
from jax.experimental.pallas import tpu_sc as plsc

_NUM_CORES = 2
_NUM_SUBCORES = 16
_NUM_WORKERS = _NUM_CORES * _NUM_SUBCORES


def kernel(x, row_embed):
    b = x.shape[0]
    w = x.shape[-2]
    d = row_embed.shape[-1]
    rows_per = w // _NUM_WORKERS

    mesh = plsc.VectorSubcoreMesh(core_axis_name="c", subcore_axis_name="s")

    @functools.partial(
        pl.kernel,
        mesh=mesh,
        out_type=jax.ShapeDtypeStruct((b * w, d), row_embed.dtype),
        scratch_types=[
            pltpu.VMEM((rows_per, d), row_embed.dtype),
            pltpu.SemaphoreType.DMA,
        ],
    )
    def _bcast(emb_hbm, out_hbm, buf, sem):
        wid = lax.axis_index("s") * _NUM_CORES + lax.axis_index("c")
        base = wid * rows_per
        pltpu.sync_copy(emb_hbm.at[pl.ds(base, rows_per)], buf)
        copies = [
            pltpu.async_copy(buf, out_hbm.at[pl.ds(bb * w + base, rows_per)], sem)
            for bb in range(b)
        ]
        for c in copies:
            c.wait()

    return _bcast(row_embed).reshape(b, w, d)


# speedup vs baseline: 1.3835x; 1.0003x over previous
import functools

import jax
import jax.numpy as jnp
from jax import lax
from jax.experimental import pallas as pl
from jax.experimental.pallas import tpu as pltpu
from jax.experimental.pallas import tpu_sc as plsc

_NUM_CORES = 2
_NUM_SUBCORES = 16
_NUM_WORKERS = _NUM_CORES * _NUM_SUBCORES


def kernel(x, row_embed):
    b = x.shape[0]
    w = x.shape[-2]
    d = row_embed.shape[-1]
    rows_per = w // _NUM_WORKERS

    mesh = plsc.VectorSubcoreMesh(core_axis_name="c", subcore_axis_name="s")

    half = rows_per // 2

    @functools.partial(
        pl.kernel,
        mesh=mesh,
        out_type=jax.ShapeDtypeStruct((b * w, d), row_embed.dtype),
        scratch_types=[
            pltpu.VMEM((half, d), row_embed.dtype),
            pltpu.VMEM((half, d), row_embed.dtype),
            pltpu.SemaphoreType.DMA,
            pltpu.SemaphoreType.DMA,
            pltpu.SemaphoreType.DMA,
        ],
    )
    def _bcast(emb_hbm, out_hbm, buf0, buf1, sem_r0, sem_r1, sem_w):
        wid = lax.axis_index("s") * _NUM_CORES + lax.axis_index("c")
        base = wid * rows_per
        r0 = pltpu.async_copy(emb_hbm.at[pl.ds(base, half)], buf0, sem_r0)
        r1 = pltpu.async_copy(emb_hbm.at[pl.ds(base + half, half)], buf1, sem_r1)
        r0.wait()
        writes = [
            pltpu.async_copy(buf0, out_hbm.at[pl.ds(bb * w + base, half)], sem_w)
            for bb in range(b)
        ]
        r1.wait()
        writes += [
            pltpu.async_copy(buf1, out_hbm.at[pl.ds(bb * w + base + half, half)], sem_w)
            for bb in range(b)
        ]
        for c in writes:
            c.wait()

    return _bcast(row_embed).reshape(b, w, d)


# speedup vs baseline: 3.1152x; 2.2517x over previous
import jax
import jax.numpy as jnp
from jax.experimental import pallas as pl


def kernel(x, row_embed):
    b = x.shape[0]
    w = x.shape[-2]
    d = row_embed.shape[-1]
    block = 2048

    def body(emb_ref, out_ref):
        out_ref[...] = jnp.broadcast_to(emb_ref[...][None], (b, block, d))

    out = pl.pallas_call(
        body,
        grid=(w // block,),
        in_specs=[pl.BlockSpec((block, d), lambda j: (j, 0))],
        out_specs=pl.BlockSpec((b, block, d), lambda j: (0, j, 0)),
        out_shape=jax.ShapeDtypeStruct((b, w, d), row_embed.dtype),
    )(row_embed)
    return out
